# initial kernel scaffold (unmeasured)
import jax
import jax.numpy as jnp
from jax import lax
from jax.experimental import pallas as pl
from jax.experimental.pallas import tpu as pltpu

N_DEV = 16
M, N = 4096, 8192
CHUNK = M // N_DEV


def kernel(x, w_mat, scale_x, scale_w):
    partial = jnp.dot(
        x.astype(jnp.bfloat16),
        w_mat.astype(jnp.bfloat16),
        preferred_element_type=jnp.float32,
    )
    scale = (scale_x.astype(jnp.float32) * scale_w.astype(jnp.float32)).reshape(1, 1)

    def body(partial_ref, scale_ref, out_ref,
             stage, acc, comm, local_sems, send_sems, recv_sems, credit_sem):
        my = lax.axis_index("i")
        right = lax.rem(my + 1, N_DEV)
        left = lax.rem(my + N_DEV - 1, N_DEV)

        def chunk_of(step):
            return lax.rem(my - 1 - step + 2 * N_DEV, N_DEV)

        ld_acc = pltpu.make_async_copy(
            partial_ref.at[pl.ds(my * CHUNK, CHUNK), :], acc, local_sems.at[0]
        )
        ld_acc.start()

        barrier = pltpu.get_barrier_semaphore()
        for nbr in (left, right):
            pl.semaphore_signal(
                barrier, inc=1, device_id=(nbr,),
                device_id_type=pl.DeviceIdType.MESH,
            )
        pl.semaphore_wait(barrier, 2)
        ld_acc.wait()

        T = 2 * (N_DEV - 1)

        def hop_send(t, src):
            slot = t % 2
            if t >= 2:
                pl.semaphore_wait(credit_sem, 1)
            rdma = pltpu.make_async_remote_copy(
                src_ref=src,
                dst_ref=comm.at[slot],
                send_sem=send_sems.at[slot],
                recv_sem=recv_sems.at[slot],
                device_id=(right,),
                device_id_type=pl.DeviceIdType.MESH,
            )
            rdma.start()
            return rdma

        def credit_left(t):
            if t < T - 2:
                pl.semaphore_signal(
                    credit_sem, inc=1, device_id=(left,),
                    device_id_type=pl.DeviceIdType.MESH,
                )

        for s in range(N_DEV - 1):
            t = s
            slot = t % 2
            rdma = hop_send(t, acc)
            ld = pltpu.make_async_copy(
                partial_ref.at[pl.ds(chunk_of(s) * CHUNK, CHUNK), :],
                stage, local_sems.at[1],
            )
            ld.start()
            rdma.wait()
            ld.wait()
            acc[:, :] = comm[slot] + stage[:, :]
            credit_left(t)

        own = lax.rem(my + 1, N_DEV)
        y = acc[:, :] * scale_ref[0, 0]
        f = y / (1.0 + jnp.exp(-jnp.clip(y, -60.0, 60.0)))
        stage[:, :] = f
        st_own = pltpu.make_async_copy(
            stage, out_ref.at[pl.ds(own * CHUNK, CHUNK), :], local_sems.at[2]
        )
        st_own.start()
        st_own.wait()

        prev_slot = None
        for s in range(N_DEV - 1):
            t = s + (N_DEV - 1)
            slot = t % 2
            src = stage if s == 0 else comm.at[prev_slot]
            rdma = hop_send(t, src)
            rdma.wait()
            c = lax.rem(my - s - 1 + 2 * N_DEV, N_DEV)
            st = pltpu.make_async_copy(
                comm.at[slot], out_ref.at[pl.ds(c * CHUNK, CHUNK), :],
                local_sems.at[2],
            )
            st.start()
            st.wait()
            credit_left(t)
            prev_slot = slot

    out = pl.pallas_call(
        body,
        out_shape=jax.ShapeDtypeStruct((M, N), jnp.float32),
        in_specs=[
            pl.BlockSpec(memory_space=pl.ANY),
            pl.BlockSpec(memory_space=pltpu.SMEM),
        ],
        out_specs=pl.BlockSpec(memory_space=pl.ANY),
        scratch_shapes=[
            pltpu.VMEM((CHUNK, N), jnp.float32),
            pltpu.VMEM((CHUNK, N), jnp.float32),
            pltpu.VMEM((2, CHUNK, N), jnp.float32),
            pltpu.SemaphoreType.DMA((3,)),
            pltpu.SemaphoreType.DMA((2,)),
            pltpu.SemaphoreType.DMA((2,)),
            pltpu.SemaphoreType.REGULAR,
        ],
        compiler_params=pltpu.CompilerParams(collective_id=0),
    )(partial, scale)
    return out


# baseline (device time: 2957794 ns/iter reference)
import jax
import jax.numpy as jnp
from jax import lax
from jax.experimental import pallas as pl
from jax.experimental.pallas import tpu as pltpu

N_DEV = 16
M, N = 4096, 8192
CHUNK = M // N_DEV


def kernel(x, w_mat, scale_x, scale_w):
    partial = jnp.dot(
        x.astype(jnp.bfloat16),
        w_mat.astype(jnp.bfloat16),
        preferred_element_type=jnp.float32,
    )
    scale = (scale_x.astype(jnp.float32) * scale_w.astype(jnp.float32)).reshape(1, 1)

    def body(partial_ref, scale_ref, out_ref,
             stage, acc, comm, local_sems, send_sems, recv_sems, credit_sem):
        my = lax.axis_index("i")
        right = lax.rem(my + 1, N_DEV)
        left = lax.rem(my + N_DEV - 1, N_DEV)

        def chunk_of(step):
            return lax.rem(my - 1 - step + 2 * N_DEV, N_DEV)

        ld_acc = pltpu.make_async_copy(
            partial_ref.at[pl.ds(my * CHUNK, CHUNK), :], acc, local_sems.at[0]
        )
        ld_acc.start()

        barrier = pltpu.get_barrier_semaphore()
        for nbr in (left, right):
            pl.semaphore_signal(
                barrier, inc=1, device_id=(nbr,),
                device_id_type=pl.DeviceIdType.MESH,
            )
        pl.semaphore_wait(barrier, 2)
        ld_acc.wait()

        T = 2 * (N_DEV - 1)

        def hop_send(t, src):
            slot = t % 2
            if t >= 2:
                pl.semaphore_wait(credit_sem, 1)
            rdma = pltpu.make_async_remote_copy(
                src_ref=src,
                dst_ref=comm.at[slot],
                send_sem=send_sems.at[slot],
                recv_sem=recv_sems.at[slot],
                device_id=(right,),
                device_id_type=pl.DeviceIdType.MESH,
            )
            rdma.start()
            return rdma

        def credit_left(t):
            if t < T - 2:
                pl.semaphore_signal(
                    credit_sem, inc=1, device_id=(left,),
                    device_id_type=pl.DeviceIdType.MESH,
                )

        for s in range(N_DEV - 1):
            t = s
            slot = t % 2
            rdma = hop_send(t, acc)
            ld = pltpu.make_async_copy(
                partial_ref.at[pl.ds(chunk_of(s) * CHUNK, CHUNK), :],
                stage, local_sems.at[1],
            )
            ld.start()
            rdma.wait()
            ld.wait()
            acc[:, :] = comm[slot] + stage[:, :]
            credit_left(t)

        own = lax.rem(my + 1, N_DEV)
        y = acc[:, :] * scale_ref[0, 0]
        f = y / (1.0 + jnp.exp(-jnp.clip(y, -60.0, 60.0)))
        stage[:, :] = f
        st_own = pltpu.make_async_copy(
            stage, out_ref.at[pl.ds(own * CHUNK, CHUNK), :], local_sems.at[2]
        )
        st_own.start()
        st_own.wait()

        prev_slot = None
        for s in range(N_DEV - 1):
            t = s + (N_DEV - 1)
            slot = t % 2
            src = stage if s == 0 else comm.at[prev_slot]
            rdma = hop_send(t, src)
            rdma.wait()
            c = lax.rem(my - s + 2 * N_DEV, N_DEV)
            st = pltpu.make_async_copy(
                comm.at[slot], out_ref.at[pl.ds(c * CHUNK, CHUNK), :],
                local_sems.at[2],
            )
            st.start()
            st.wait()
            credit_left(t)
            prev_slot = slot

    out = pl.pallas_call(
        body,
        out_shape=jax.ShapeDtypeStruct((M, N), jnp.float32),
        in_specs=[
            pl.BlockSpec(memory_space=pl.ANY),
            pl.BlockSpec(memory_space=pltpu.SMEM),
        ],
        out_specs=pl.BlockSpec(memory_space=pl.ANY),
        scratch_shapes=[
            pltpu.VMEM((CHUNK, N), jnp.float32),
            pltpu.VMEM((CHUNK, N), jnp.float32),
            pltpu.VMEM((2, CHUNK, N), jnp.float32),
            pltpu.SemaphoreType.DMA((3,)),
            pltpu.SemaphoreType.DMA((2,)),
            pltpu.SemaphoreType.DMA((2,)),
            pltpu.SemaphoreType.REGULAR,
        ],
        compiler_params=pltpu.CompilerParams(collective_id=0),
    )(partial, scale)
    return out


# device time: 1646319 ns/iter; 1.7966x vs baseline; 1.7966x over previous
import jax
import jax.numpy as jnp
from jax import lax
from jax.experimental import pallas as pl
from jax.experimental.pallas import tpu as pltpu

N_DEV = 16
M, N = 4096, 8192
CHUNK = M // N_DEV
HALF = N // 2
T = 2 * (N_DEV - 1)


class _Dir:

    def __init__(self, sign, peer, credit_peer, col_off,
                 stage, acc, comm, lsems, send_sems, recv_sems, credit):
        self.sign = sign
        self.peer = peer
        self.credit_peer = credit_peer
        self.col_off = col_off
        self.stage = stage
        self.acc = acc
        self.comm = comm
        self.lsems = lsems
        self.send_sems = send_sems
        self.recv_sems = recv_sems
        self.credit = credit


def kernel(x, w_mat, scale_x, scale_w):
    partial = jnp.dot(
        x.astype(jnp.bfloat16),
        w_mat.astype(jnp.bfloat16),
        preferred_element_type=jnp.float32,
    )
    scale = (scale_x.astype(jnp.float32) * scale_w.astype(jnp.float32)).reshape(1, 1)

    def body(partial_ref, scale_ref, out_ref,
             stage0, acc0, comm0, stage1, acc1, comm1,
             lsems0, lsems1, send0, recv0, send1, recv1, cred0, cred1):
        my = lax.axis_index("i")
        right = lax.rem(my + 1, N_DEV)
        left = lax.rem(my + N_DEV - 1, N_DEV)

        def modc(v):
            return lax.rem(v + 4 * N_DEV, N_DEV)

        dirs = [
            _Dir(+1, right, left, 0,
                 stage0, acc0, comm0, lsems0, send0, recv0, cred0),
            _Dir(-1, left, right, HALF,
                 stage1, acc1, comm1, lsems1, send1, recv1, cred1),
        ]

        own_loads = []
        for d in dirs:
            ld = pltpu.make_async_copy(
                partial_ref.at[pl.ds(my * CHUNK, CHUNK),
                               pl.ds(d.col_off, HALF)],
                d.acc, d.lsems.at[0],
            )
            ld.start()
            own_loads.append(ld)

        barrier = pltpu.get_barrier_semaphore()
        for nbr in (left, right):
            pl.semaphore_signal(
                barrier, inc=1, device_id=(nbr,),
                device_id_type=pl.DeviceIdType.MESH,
            )
        pl.semaphore_wait(barrier, 2)
        for ld in own_loads:
            ld.wait()

        def start_send(d, t, src):
            slot = t % 2
            if t >= 2:
                pl.semaphore_wait(d.credit, 1)
            rdma = pltpu.make_async_remote_copy(
                src_ref=src,
                dst_ref=d.comm.at[slot],
                send_sem=d.send_sems.at[slot],
                recv_sem=d.recv_sems.at[slot],
                device_id=(d.peer,),
                device_id_type=pl.DeviceIdType.MESH,
            )
            rdma.start()
            return rdma

        def credit_upstream(d, t):
            if t < T - 2:
                pl.semaphore_signal(
                    d.credit, inc=1, device_id=(d.credit_peer,),
                    device_id_type=pl.DeviceIdType.MESH,
                )

        for s in range(N_DEV - 1):
            t = s
            slot = t % 2
            rdmas = []
            loads = []
            for d in dirs:
                rdmas.append(start_send(d, t, d.acc))
                c = modc(my - d.sign * (1 + s))
                ld = pltpu.make_async_copy(
                    partial_ref.at[pl.ds(c * CHUNK, CHUNK),
                                   pl.ds(d.col_off, HALF)],
                    d.stage, d.lsems.at[0],
                )
                ld.start()
                loads.append(ld)
            for r in rdmas:
                r.wait()
            for ld in loads:
                ld.wait()
            for d in dirs:
                d.acc[:, :] = d.comm[slot] + d.stage[:, :]
            for d in dirs:
                credit_upstream(d, t)

        own_stores = []
        for d in dirs:
            own = modc(my + d.sign)
            y = d.acc[:, :] * scale_ref[0, 0]
            f = y / (1.0 + jnp.exp(-jnp.clip(y, -60.0, 60.0)))
            d.stage[:, :] = f
            st = pltpu.make_async_copy(
                d.stage,
                out_ref.at[pl.ds(own * CHUNK, CHUNK), pl.ds(d.col_off, HALF)],
                d.lsems.at[1],
            )
            st.start()
            own_stores.append(st)
        for st in own_stores:
            st.wait()

        prev_slot = None
        for s in range(N_DEV - 1):
            t = s + (N_DEV - 1)
            slot = t % 2
            rdmas = []
            for d in dirs:
                src = d.stage if s == 0 else d.comm.at[prev_slot]
                rdmas.append(start_send(d, t, src))
            for r in rdmas:
                r.wait()
            stores = []
            for d in dirs:
                c = modc(my - d.sign * s)
                st = pltpu.make_async_copy(
                    d.comm.at[slot],
                    out_ref.at[pl.ds(c * CHUNK, CHUNK),
                               pl.ds(d.col_off, HALF)],
                    d.lsems.at[1],
                )
                st.start()
                stores.append(st)
            for st in stores:
                st.wait()
            for d in dirs:
                credit_upstream(d, t)
            prev_slot = slot

    out = pl.pallas_call(
        body,
        out_shape=jax.ShapeDtypeStruct((M, N), jnp.float32),
        in_specs=[
            pl.BlockSpec(memory_space=pl.ANY),
            pl.BlockSpec(memory_space=pltpu.SMEM),
        ],
        out_specs=pl.BlockSpec(memory_space=pl.ANY),
        scratch_shapes=[
            pltpu.VMEM((CHUNK, HALF), jnp.float32),
            pltpu.VMEM((CHUNK, HALF), jnp.float32),
            pltpu.VMEM((2, CHUNK, HALF), jnp.float32),
            pltpu.VMEM((CHUNK, HALF), jnp.float32),
            pltpu.VMEM((CHUNK, HALF), jnp.float32),
            pltpu.VMEM((2, CHUNK, HALF), jnp.float32),
            pltpu.SemaphoreType.DMA((2,)),
            pltpu.SemaphoreType.DMA((2,)),
            pltpu.SemaphoreType.DMA((2,)),
            pltpu.SemaphoreType.DMA((2,)),
            pltpu.SemaphoreType.DMA((2,)),
            pltpu.SemaphoreType.DMA((2,)),
            pltpu.SemaphoreType.REGULAR,
            pltpu.SemaphoreType.REGULAR,
        ],
        compiler_params=pltpu.CompilerParams(collective_id=0),
    )(partial, scale)
    return out


# device time: 1593507 ns/iter; 1.8562x vs baseline; 1.0331x over previous
import jax
import jax.numpy as jnp
from jax import lax
from jax.experimental import pallas as pl
from jax.experimental.pallas import tpu as pltpu

N_DEV = 16
M, N = 4096, 8192
CHUNK = M // N_DEV
QUART = N // 4
T = 2 * (N_DEV - 1)
WAIT_ORDER = (0, 2, 1, 3)


class _Stream:
    def __init__(self, sign, peer, credit_peer, col_off,
                 stage, acc, comm, lsems, send_sems, recv_sems, credit):
        self.sign = sign
        self.peer = peer
        self.credit_peer = credit_peer
        self.col_off = col_off
        self.stage = stage
        self.acc = acc
        self.comm = comm
        self.lsems = lsems
        self.send_sems = send_sems
        self.recv_sems = recv_sems
        self.credit = credit
        self.prev_st = None


def kernel(x, w_mat, scale_x, scale_w):
    partial = jnp.dot(
        x.astype(jnp.bfloat16),
        w_mat.astype(jnp.bfloat16),
        preferred_element_type=jnp.float32,
    )
    scale = (scale_x.astype(jnp.float32) * scale_w.astype(jnp.float32)).reshape(1, 1)

    def body(partial_ref, scale_ref, out_ref, *scratch):
        my = lax.axis_index("i")
        right = lax.rem(my + 1, N_DEV)
        left = lax.rem(my + N_DEV - 1, N_DEV)

        def modc(v):
            return lax.rem(v + 4 * N_DEV, N_DEV)

        streams = []
        for k in range(4):
            sign = +1 if k < 2 else -1
            streams.append(_Stream(
                sign,
                right if sign > 0 else left,
                left if sign > 0 else right,
                k * QUART,
                scratch[3 * k], scratch[3 * k + 1], scratch[3 * k + 2],
                scratch[12 + k], scratch[16 + k], scratch[20 + k],
                scratch[24 + k],
            ))

        own_loads = []
        for d in streams:
            ld = pltpu.make_async_copy(
                partial_ref.at[pl.ds(my * CHUNK, CHUNK),
                               pl.ds(d.col_off, QUART)],
                d.acc, d.lsems.at[0],
            )
            ld.start()
            own_loads.append(ld)

        barrier = pltpu.get_barrier_semaphore()
        for nbr in (left, right):
            pl.semaphore_signal(
                barrier, inc=1, device_id=(nbr,),
                device_id_type=pl.DeviceIdType.MESH,
            )
        pl.semaphore_wait(barrier, 2)
        for ld in own_loads:
            ld.wait()

        def start_send(d, t, src):
            slot = t % 2
            if t >= 2:
                pl.semaphore_wait(d.credit, 1)
            rdma = pltpu.make_async_remote_copy(
                src_ref=src,
                dst_ref=d.comm.at[slot],
                send_sem=d.send_sems.at[slot],
                recv_sem=d.recv_sems.at[slot],
                device_id=(d.peer,),
                device_id_type=pl.DeviceIdType.MESH,
            )
            rdma.start()
            return rdma

        def credit_upstream(d):
            pl.semaphore_signal(
                d.credit, inc=1, device_id=(d.credit_peer,),
                device_id_type=pl.DeviceIdType.MESH,
            )

        for s in range(N_DEV - 1):
            t = s
            slot = t % 2
            rdmas = []
            loads = []
            for d in streams:
                rdmas.append(start_send(d, t, d.acc))
            for d in streams:
                c = modc(my - d.sign * (1 + s))
                ld = pltpu.make_async_copy(
                    partial_ref.at[pl.ds(c * CHUNK, CHUNK),
                                   pl.ds(d.col_off, QUART)],
                    d.stage, d.lsems.at[0],
                )
                ld.start()
                loads.append(ld)
            for k in WAIT_ORDER:
                d = streams[k]
                rdmas[k].wait()
                loads[k].wait()
                d.acc[:, :] = d.comm[slot] + d.stage[:, :]
            if t < T - 2:
                for d in streams:
                    credit_upstream(d)

        own_stores = []
        for d in streams:
            own = modc(my + d.sign)
            y = d.acc[:, :] * scale_ref[0, 0]
            f = y / (1.0 + jnp.exp(-jnp.clip(y, -60.0, 60.0)))
            d.stage[:, :] = f
            st = pltpu.make_async_copy(
                d.stage,
                out_ref.at[pl.ds(own * CHUNK, CHUNK),
                           pl.ds(d.col_off, QUART)],
                d.lsems.at[1],
            )
            st.start()
            own_stores.append(st)
        for st in own_stores:
            st.wait()

        prev_slot = None
        for s in range(N_DEV - 1):
            t = s + (N_DEV - 1)
            slot = t % 2
            rdmas = []
            for d in streams:
                src = d.stage if s == 0 else d.comm.at[prev_slot]
                rdmas.append(start_send(d, t, src))
            for k in WAIT_ORDER:
                d = streams[k]
                rdmas[k].wait()
                if s > 0:
                    d.prev_st.wait()
                    if s + (N_DEV - 1) - 1 < T - 2:
                        credit_upstream(d)
                c = modc(my - d.sign * s)
                st = pltpu.make_async_copy(
                    d.comm.at[slot],
                    out_ref.at[pl.ds(c * CHUNK, CHUNK),
                               pl.ds(d.col_off, QUART)],
                    d.lsems.at[1],
                )
                st.start()
                d.prev_st = st
            prev_slot = slot
        for d in streams:
            d.prev_st.wait()

    out = pl.pallas_call(
        body,
        out_shape=jax.ShapeDtypeStruct((M, N), jnp.float32),
        in_specs=[
            pl.BlockSpec(memory_space=pl.ANY),
            pl.BlockSpec(memory_space=pltpu.SMEM),
        ],
        out_specs=pl.BlockSpec(memory_space=pl.ANY),
        scratch_shapes=(
            [buf
             for _ in range(4)
             for buf in (pltpu.VMEM((CHUNK, QUART), jnp.float32),
                         pltpu.VMEM((CHUNK, QUART), jnp.float32),
                         pltpu.VMEM((2, CHUNK, QUART), jnp.float32))]
            + [pltpu.SemaphoreType.DMA((2,)) for _ in range(4)]
            + [pltpu.SemaphoreType.DMA((2,)) for _ in range(4)]
            + [pltpu.SemaphoreType.DMA((2,)) for _ in range(4)]
            + [pltpu.SemaphoreType.REGULAR for _ in range(4)]
        ),
        compiler_params=pltpu.CompilerParams(collective_id=0),
    )(partial, scale)
    return out


# device time: 1496024 ns/iter; 1.9771x vs baseline; 1.0652x over previous
import jax
import jax.numpy as jnp
from jax import lax
from jax.experimental import pallas as pl
from jax.experimental.pallas import tpu as pltpu

N_DEV = 16
M, N = 4096, 8192
CHUNK = M // N_DEV
QUART = N // 4
T = 2 * (N_DEV - 1)
ORDER = (0, 2, 1, 3)


class _Stream:
    def __init__(self, sign, peer, credit_peer, col_off,
                 stage, acc, comm, lsems, send_sems, recv_sems, credit):
        self.sign = sign
        self.peer = peer
        self.credit_peer = credit_peer
        self.col_off = col_off
        self.stage = stage
        self.acc = acc
        self.comm = comm
        self.lsems = lsems
        self.send_sems = send_sems
        self.recv_sems = recv_sems
        self.credit = credit
        self.rdma = None
        self.load = None
        self.st_prev = None


def kernel(x, w_mat, scale_x, scale_w):
    partial = jnp.dot(
        x.astype(jnp.bfloat16),
        w_mat.astype(jnp.bfloat16),
        preferred_element_type=jnp.float32,
    )
    scale = (scale_x.astype(jnp.float32) * scale_w.astype(jnp.float32)).reshape(1, 1)

    def body(partial_ref, scale_ref, out_ref, *scratch):
        my = lax.axis_index("i")
        right = lax.rem(my + 1, N_DEV)
        left = lax.rem(my + N_DEV - 1, N_DEV)

        def modc(v):
            return lax.rem(v + 4 * N_DEV, N_DEV)

        streams = []
        for k in range(4):
            sign = +1 if k < 2 else -1
            streams.append(_Stream(
                sign,
                right if sign > 0 else left,
                left if sign > 0 else right,
                k * QUART,
                scratch[3 * k], scratch[3 * k + 1], scratch[3 * k + 2],
                scratch[12 + k], scratch[16 + k], scratch[20 + k],
                scratch[24 + k],
            ))

        def start_send(d, t, src):
            slot = t % 2
            if t >= 2:
                pl.semaphore_wait(d.credit, 1)
            rdma = pltpu.make_async_remote_copy(
                src_ref=src,
                dst_ref=d.comm.at[slot],
                send_sem=d.send_sems.at[slot],
                recv_sem=d.recv_sems.at[slot],
                device_id=(d.peer,),
                device_id_type=pl.DeviceIdType.MESH,
            )
            rdma.start()
            return rdma

        def start_load(d, s):
            c = modc(my - d.sign * (1 + s))
            ld = pltpu.make_async_copy(
                partial_ref.at[pl.ds(c * CHUNK, CHUNK),
                               pl.ds(d.col_off, QUART)],
                d.stage, d.lsems.at[0],
            )
            ld.start()
            return ld

        def credit_upstream(d):
            pl.semaphore_signal(
                d.credit, inc=1, device_id=(d.credit_peer,),
                device_id_type=pl.DeviceIdType.MESH,
            )

        own_loads = []
        for d in streams:
            ld = pltpu.make_async_copy(
                partial_ref.at[pl.ds(my * CHUNK, CHUNK),
                               pl.ds(d.col_off, QUART)],
                d.acc, d.lsems.at[0],
            )
            ld.start()
            own_loads.append(ld)

        barrier = pltpu.get_barrier_semaphore()
        for nbr in (left, right):
            pl.semaphore_signal(
                barrier, inc=1, device_id=(nbr,),
                device_id_type=pl.DeviceIdType.MESH,
            )
        pl.semaphore_wait(barrier, 2)
        for ld in own_loads:
            ld.wait()

        for k in ORDER:
            d = streams[k]
            d.rdma = start_send(d, 0, d.acc)
            d.load = start_load(d, 0)
        for s in range(1, N_DEV - 1):
            for k in ORDER:
                d = streams[k]
                d.rdma.wait()
                d.load.wait()
                d.acc[:, :] = d.comm[(s - 1) % 2] + d.stage[:, :]
                credit_upstream(d)
                d.rdma = start_send(d, s, d.acc)
                d.load = start_load(d, s)

        last_slot = (N_DEV - 2) % 2
        for k in ORDER:
            d = streams[k]
            d.rdma.wait()
            d.load.wait()
            d.acc[:, :] = d.comm[last_slot] + d.stage[:, :]
            own = modc(my + d.sign)
            y = d.acc[:, :] * scale_ref[0, 0]
            d.stage[:, :] = y / (1.0 + jnp.exp(-jnp.clip(y, -60.0, 60.0)))
            st = pltpu.make_async_copy(
                d.stage,
                out_ref.at[pl.ds(own * CHUNK, CHUNK),
                           pl.ds(d.col_off, QUART)],
                d.lsems.at[1],
            )
            st.start()
            d.st_prev = st
            d.rdma = start_send(d, N_DEV - 1, d.stage)

        for s in range(1, N_DEV):
            t_prev = (N_DEV - 1) + s - 1
            q = t_prev % 2
            for k in ORDER:
                d = streams[k]
                d.rdma.wait()
                d.st_prev.wait()
                if t_prev - 1 < T - 2:
                    credit_upstream(d)
                if s < N_DEV - 1:
                    d.rdma = start_send(d, t_prev + 1, d.comm.at[q])
                c = modc(my - d.sign * (s - 1))
                st = pltpu.make_async_copy(
                    d.comm.at[q],
                    out_ref.at[pl.ds(c * CHUNK, CHUNK),
                               pl.ds(d.col_off, QUART)],
                    d.lsems.at[1],
                )
                st.start()
                d.st_prev = st
        for d in streams:
            d.st_prev.wait()

    out = pl.pallas_call(
        body,
        out_shape=jax.ShapeDtypeStruct((M, N), jnp.float32),
        in_specs=[
            pl.BlockSpec(memory_space=pl.ANY),
            pl.BlockSpec(memory_space=pltpu.SMEM),
        ],
        out_specs=pl.BlockSpec(memory_space=pl.ANY),
        scratch_shapes=(
            [buf
             for _ in range(4)
             for buf in (pltpu.VMEM((CHUNK, QUART), jnp.float32),
                         pltpu.VMEM((CHUNK, QUART), jnp.float32),
                         pltpu.VMEM((2, CHUNK, QUART), jnp.float32))]
            + [pltpu.SemaphoreType.DMA((2,)) for _ in range(4)]
            + [pltpu.SemaphoreType.DMA((2,)) for _ in range(4)]
            + [pltpu.SemaphoreType.DMA((2,)) for _ in range(4)]
            + [pltpu.SemaphoreType.REGULAR for _ in range(4)]
        ),
        compiler_params=pltpu.CompilerParams(collective_id=0),
    )(partial, scale)
    return out
